# baseline (speedup 1.0000x reference)
import jax
import jax.numpy as jnp
from jax import lax
from jax.experimental import pallas as pl
from jax.experimental.pallas import tpu as pltpu

H = 375
W = 1242
ROWS = H * 10
BUF_ROWS = ROWS + 10
DUMMY_FLAT = 469 * 1024
CH = 8192
CR, CL = 64, CH // 64
NS = 4
UNROLL = 8
IMG_MEAN = (0.485, 0.456, 0.406)
IMG_STD = (0.229, 0.224, 0.225)


def _bf_jnp(x):
    b = lax.bitcast_convert_type(jnp.asarray(x, jnp.float32), jnp.int32)
    r = (b + 32767 + ((b >> 16) & 1)) & jnp.int32(-65536)
    return lax.bitcast_convert_type(r, jnp.float32)


def _bf_pl(x):
    b = pltpu.bitcast(x, jnp.int32)
    r = (b + 32767 + ((b >> 16) & 1)) & jnp.int32(-65536)
    return pltpu.bitcast(r, jnp.float32)


def _prologue_kernel(pred_ref, tgt_ref, rrect_ref, rt_ref, out_ref):
    nb = pred_ref.shape[0]
    q = pred_ref[:, 0:4]
    s = jnp.sum(q * q, axis=1, keepdims=True)
    qn = q * lax.rsqrt(s)
    w, x, y, z = (qn[:, 0:1], qn[:, 1:2], qn[:, 2:3], qn[:, 3:4])
    rp = [
        [1 - 2 * (y * y + z * z), 2 * (x * y - w * z), 2 * (x * z + w * y)],
        [2 * (x * y + w * z), 1 - 2 * (x * x + z * z), 2 * (y * z - w * x)],
        [2 * (x * z - w * y), 2 * (y * z + w * x), 1 - 2 * (x * x + y * y)],
    ]
    rp = [[_bf_pl(e) for e in row] for row in rp]
    rg = [[_bf_pl(tgt_ref[:, 4 * i + j:4 * i + j + 1]) for j in range(3)]
          for i in range(3)]
    acc = None
    for i in range(3):
        for k in range(3):
            a = rp[0][i] * rg[0][k] + rp[1][i] * rg[1][k] + rp[2][i] * rg[2][k]
            d = a - (1.0 if i == k else 0.0)
            acc = d * d if acc is None else acc + d * d
    out_ref[:, 15:16] = jnp.sqrt(acc)
    tp = _bf_pl(pred_ref[:, 4:7])
    for i in range(3):
        for j in range(3):
            out_ref[:, 4 * i + j:4 * i + j + 1] = rg[i][j] - rp[i][j]
        out_ref[:, 4 * i + 3:4 * i + 4] = (
            _bf_pl(tgt_ref[:, 4 * i + 3:4 * i + 4]) - tp[:, i:i + 1])
    out_ref[:, 12:15] = tp
    m = None
    for j in range(4):
        t = rrect_ref[:, j:j + 1] * rt_ref[j:j + 1, :]
        m = t if m is None else m + t
    m = _bf_pl(m)
    for i in range(3):
        for j in range(4):
            out_ref[:, 16 + 4 * i + j:17 + 4 * i + j] = jnp.broadcast_to(
                m[i:i + 1, j:j + 1], (nb, 1))


def _main_kernel(xs_ref, ys_ref, zs_ref, it_ref, tab_ref, p_ref, size_ref,
                 img_ref, ms_ref,
                 fV, dV, iV, fS, dS, iS, accs,
                 d0, d1, d2, d3, i0, i1, i2, i3, sems):
    b = pl.program_id(0)
    c = pl.program_id(1)
    nc = pl.num_programs(1)
    dbufs = (d0, d1, d2, d3)
    ibufs = (i0, i1, i2, i3)

    @pl.when(c == 0)
    def _():
        zero = jnp.zeros((BUF_ROWS, 128), jnp.float32)
        for buf in dbufs + ibufs:
            buf[...] = zero
        accs[...] = jnp.zeros((CR, CL), jnp.float32)

    X = _bf_pl(xs_ref[0, 0])
    Y = _bf_pl(ys_ref[0, 0])
    Z = _bf_pl(zs_ref[0, 0])
    IT = it_ref[0, 0]

    def t(col):
        return tab_ref[b, col]

    pb = []
    for i in range(3):
        pb.append(_bf_pl(t(16 + 4 * i) * X + t(17 + 4 * i) * Y
                         + t(18 + 4 * i) * Z + t(19 + 4 * i)))
    l1 = None
    for i in range(3):
        term = jnp.abs(t(4 * i) * pb[0] + t(4 * i + 1) * pb[1]
                       + t(4 * i + 2) * pb[2] + t(4 * i + 3))
        l1 = term if l1 is None else l1 + term
    rc0 = pb[0] + t(12)
    rc1 = pb[1] + t(13)
    rc2 = pb[2] + t(14)
    rcb0 = _bf_pl(rc0)
    rcb1 = _bf_pl(rc1)
    rcb2 = _bf_pl(rc2)
    pr = []
    for i in range(3):
        pr.append(p_ref[0, 4 * i] * rcb0 + p_ref[0, 4 * i + 1] * rcb1
                  + p_ref[0, 4 * i + 2] * rcb2 + p_ref[0, 4 * i + 3])
    u = pr[0] / pr[2]
    v = pr[1] / pr[2]
    jj = (c * CH + lax.broadcasted_iota(jnp.int32, (CR, CL), 0) * CL
          + lax.broadcasted_iota(jnp.int32, (CR, CL), 1))
    valid = jj < size_ref[b]
    mask = ((u >= 0) & (u < W) & (v >= 0) & (v < H) & (rc2 > 0) & valid)
    ui = u.astype(jnp.int32)
    vi = v.astype(jnp.int32)
    flat = jnp.where(mask, vi * 1280 + ui, DUMMY_FLAT)
    accs[...] = accs[...] + jnp.where(valid, l1, 0.0)
    fV[...] = flat
    dV[...] = rc2
    iV[...] = IT

    cps = [pltpu.make_async_copy(fV, fS, sems.at[0]),
           pltpu.make_async_copy(dV, dS, sems.at[1]),
           pltpu.make_async_copy(iV, iS, sems.at[2])]
    for cp in cps:
        cp.start()
    for cp in cps:
        cp.wait()

    flat_iota = (lax.broadcasted_iota(jnp.int32, (8, 128), 0) * 128
                 + lax.broadcasted_iota(jnp.int32, (8, 128), 1))

    def body(g, carry):
        r = lax.shift_right_logical(g, 4)
        colbase = lax.bitwise_and(lax.shift_left(g, 3), 127)
        for k in range(UNROLL):
            f = fS[r, colbase + k]
            cbase = pl.multiple_of(
                lax.shift_left(lax.shift_right_logical(f, 10), 3), 8)
            off = lax.bitwise_and(f, 1023)
            m = flat_iota == off
            db = dbufs[k % NS]
            ib = ibufs[k % NS]
            cur = db[pl.ds(cbase, 8), :]
            db[pl.ds(cbase, 8), :] = jnp.maximum(
                cur, jnp.where(m, dS[r, colbase + k], 0.0))
            cur2 = ib[pl.ds(cbase, 8), :]
            ib[pl.ds(cbase, 8), :] = jnp.maximum(
                cur2, jnp.where(m, iS[r, colbase + k], 0.0))
        return carry

    lax.fori_loop(0, CH // UNROLL, body, 0)

    @pl.when(c == nc - 1)
    def _():
        dm = jnp.maximum(jnp.maximum(d0[...], d1[...]),
                         jnp.maximum(d2[...], d3[...]))[0:ROWS, :]
        im = jnp.maximum(jnp.maximum(i0[...], i1[...]),
                         jnp.maximum(i2[...], i3[...]))[0:ROWS, :]
        img_ref[0, 0] = (dm - IMG_MEAN[0]) * (1.0 / IMG_STD[0])
        img_ref[0, 1] = (dm - IMG_MEAN[1]) * (1.0 / IMG_STD[1])
        img_ref[0, 2] = (im - IMG_MEAN[2]) * (1.0 / IMG_STD[2])
        s0 = jnp.sum(accs[...], axis=0, keepdims=True)
        s1 = jnp.sum(s0, axis=1, keepdims=True)
        ms_ref[0, 0:1, :] = jnp.broadcast_to(s1, (1, 128))


def kernel(predT, srcClrT, srcDepthT, ptCldT, ptCldSize, targetTransformT,
           P_rect, R_rect, RT):
    B, N = ptCldT.shape[0], ptCldT.shape[1]
    C = N // CH
    f32 = jnp.float32
    tab = pl.pallas_call(
        _prologue_kernel,
        out_shape=jax.ShapeDtypeStruct((B, 128), f32),
    )(predT, targetTransformT.reshape(B, 16), R_rect, RT)

    xs = ptCldT[:, :, 0].reshape(B, C, CR, CL)
    ys = ptCldT[:, :, 1].reshape(B, C, CR, CL)
    zs = ptCldT[:, :, 2].reshape(B, C, CR, CL)
    it = ptCldT[:, :, 3].reshape(B, C, CR, CL)
    p12 = _bf_jnp(P_rect).reshape(1, 12)

    blk = pl.BlockSpec((1, 1, CR, CL), lambda b, c: (b, c, 0, 0))
    smem = pl.BlockSpec(memory_space=pltpu.SMEM)
    img, ms = pl.pallas_call(
        _main_kernel,
        grid=(B, C),
        in_specs=[blk, blk, blk, blk, smem, smem, smem],
        out_specs=[
            pl.BlockSpec((1, 3, ROWS, 128), lambda b, c: (b, 0, 0, 0)),
            pl.BlockSpec((1, 1, 128), lambda b, c: (b, 0, 0)),
        ],
        out_shape=[
            jax.ShapeDtypeStruct((B, 3, ROWS, 128), f32),
            jax.ShapeDtypeStruct((B, 1, 128), f32),
        ],
        scratch_shapes=(
            [pltpu.VMEM((CR, CL), jnp.int32),
             pltpu.VMEM((CR, CL), f32),
             pltpu.VMEM((CR, CL), f32),
             pltpu.SMEM((CR, CL), jnp.int32),
             pltpu.SMEM((CR, CL), f32),
             pltpu.SMEM((CR, CL), f32),
             pltpu.VMEM((CR, CL), f32)]
            + [pltpu.VMEM((BUF_ROWS, 128), f32) for _ in range(8)]
            + [pltpu.SemaphoreType.DMA((3,))]),
        compiler_params=pltpu.CompilerParams(
            dimension_semantics=("parallel", "arbitrary"),
            vmem_limit_bytes=56 * 1024 * 1024,
        ),
    )(xs, ys, zs, it, tab, p12, ptCldSize)

    rotationLoss = tab[:, 15]
    manhattanLoss = jnp.mean(ms[:, 0, 0] / ptCldSize.astype(f32))
    totalLoss = (manhattanLoss + rotationLoss).astype(f32)
    img_full = img.reshape(B, 3, H, 1280)[:, :, :, :W]
    return totalLoss, manhattanLoss, img_full
